# baseline (speedup 1.0000x reference)
import functools

import jax
import jax.numpy as jnp
from jax import lax
from jax.experimental import pallas as pl
from jax.experimental.pallas import tpu as pltpu
from jax.experimental.pallas import tpu_sc as plsc

N = 10000
E = 160000
B = 256
FXD = 78
HEADS = 10
D_GAT = FXD * HEADS
EMBED = 320
OUT_DIM = 128

NP = 10112
EP = 172032
NWORK = 32
EW = EP // NWORK
CH = 128
NCH = EW // CH
ROWS = NP // 16
NCH4 = EP // 16 // CH
EW4 = EP // 16
HPASS = HEADS // 2
FP = 80
D_PAD = HEADS * FP
DGP = 800
PF4 = 32
NPL = 25
PPASS = 13
D_GC = 832
PW6 = 64
NPL6 = 13
RB = 632
GRID = NP // RB
GPW = B // NWORK


def _sc_mesh():
    return plsc.VectorSubcoreMesh(core_axis_name="c", subcore_axis_name="s")


def _sc_params():
    import dataclasses
    cp = pltpu.CompilerParams(use_tc_tiling_on_sc=False)
    if "needs_layout_passes" in pltpu.CompilerParams.__dataclass_fields__:
        cp = dataclasses.replace(cp, needs_layout_passes=False)
    return dict(mesh=_sc_mesh(), compiler_params=cp)


def _k1_body(x_ref, w3_ref, asx_ref, adx_ref, h_ref, a_ref, b_ref):
    x = x_ref[...]
    for p in range(NPL):
        h_ref[p] = jnp.dot(x, w3_ref[p], preferred_element_type=jnp.float32)
    a_ref[...] = jnp.dot(x, asx_ref[...], preferred_element_type=jnp.float32)
    b_ref[...] = jnp.dot(x, adx_ref[...], preferred_element_type=jnp.float32)


def _k1(xp, w3, asx, adx):
    return pl.pallas_call(
        _k1_body,
        grid=(GRID,),
        in_specs=[
            pl.BlockSpec((RB, FXD), lambda i: (i, 0)),
            pl.BlockSpec((NPL, FXD, PF4), lambda i: (0, 0, 0)),
            pl.BlockSpec((FXD, 16), lambda i: (0, 0)),
            pl.BlockSpec((FXD, 16), lambda i: (0, 0)),
        ],
        out_specs=[
            pl.BlockSpec((NPL, RB, PF4), lambda i: (0, i, 0)),
            pl.BlockSpec((RB, 16), lambda i: (i, 0)),
            pl.BlockSpec((RB, 16), lambda i: (i, 0)),
        ],
        out_shape=[
            jax.ShapeDtypeStruct((NPL, NP, PF4), jnp.float32),
            jax.ShapeDtypeStruct((NP, 16), jnp.float32),
            jax.ShapeDtypeStruct((NP, 16), jnp.float32),
        ],
    )(xp, w3, asx, adx)


def _k2(*args):
    return pl.kernel(
        _k2_body,
        out_type=jax.ShapeDtypeStruct((EP, 16), jnp.float32),
        scratch_types=[
            pltpu.VMEM((NCH, CH), jnp.int32),
            pltpu.VMEM((NCH, CH), jnp.int32),
            pltpu.VMEM((CH, 16), jnp.float32),
            pltpu.VMEM((CH, 16), jnp.float32),
            pltpu.VMEM((CH, 16), jnp.float32),
        ],
        **_sc_params(),
    )(*args)


def _k2_body(src_hbm, dst_hbm, a_hbm, b_hbm, ex_hbm,
        sidx, didx, arow, brow, exrow):
    cid = lax.axis_index("c")
    sid = lax.axis_index("s")
    wid = sid * 2 + cid

    pltpu.sync_copy(src_hbm.at[wid], sidx)
    pltpu.sync_copy(dst_hbm.at[wid], didx)
    lanes = lax.iota(jnp.int32, 16)

    @pl.loop(0, NCH)
    def _(j):
        pltpu.sync_copy(a_hbm.at[sidx.at[j]], arow)
        pltpu.sync_copy(b_hbm.at[didx.at[j]], brow)

        @pl.loop(0, CH)
        def _(i):
            v = arow[i] + brow[i]
            al = jnp.where(v >= 0.0, v, 0.2 * v)
            ex = jnp.exp(al)
            exrow[i] = jnp.where(
                lanes < HEADS, ex, jnp.where(lanes == HEADS, 1.0, 0.0))

        pltpu.sync_copy(exrow, ex_hbm.at[pl.ds(wid * EW + j * CH, CH)])


def _bcast_lane(v16, idx):
    return lax.gather(
        v16, idx[:, None],
        lax.GatherDimensionNumbers(offset_dims=(), collapsed_slice_dims=(0,),
                                   start_index_map=(0,)),
        (1,), mode=lax.GatherScatterMode.PROMISE_IN_BOUNDS)


def _k4(*args):
    return pl.kernel(
        _k4_body,
        out_type=(jax.ShapeDtypeStruct((NPL, NP, PF4), jnp.float32),
                  jax.ShapeDtypeStruct((NP, 16), jnp.float32)),
        scratch_types=[
            pltpu.VMEM((NCH4, CH), jnp.int32),
            pltpu.VMEM((NCH4, CH), jnp.int32),
            pltpu.VMEM((2, CH, PF4), jnp.float32),
            pltpu.VMEM((2, CH, PF4), jnp.float32),
            pltpu.VMEM((2 * CH, 16), jnp.float32),
            pltpu.VMEM((ROWS, PF4), jnp.float32),
            pltpu.VMEM((ROWS, 16), jnp.float32),
            pltpu.VMEM_SHARED((NP, PF4), jnp.float32),
            pltpu.VMEM_SHARED((NP, 16), jnp.float32),
            pltpu.SemaphoreType.DMA,
        ],
        **_sc_params(),
    )(*args)


def _k4_body(h_hbm, src_hbm, dst_hbm, ex_hbm, agg_hbm, den_hbm,
        sidx, didx, hrow, srow, exbuf, zbuf, zbuf16, acc_sh, den_sh, gsem):
    cid = lax.axis_index("c")
    sid = lax.axis_index("s")

    @pl.loop(0, ROWS)
    def _(r):
        for q in range(PF4 // 16):
            zbuf[r, pl.ds(q * 16, 16)] = jnp.zeros((16,), jnp.float32)
        zbuf16[r] = jnp.zeros((16,), jnp.float32)

    pltpu.sync_copy(src_hbm.at[sid], sidx)
    pltpu.sync_copy(dst_hbm.at[sid], didx)
    pltpu.sync_copy(zbuf16, den_sh.at[pl.ds(sid * ROWS, ROWS)])

    lanes = lax.iota(jnp.int32, 16)

    @pl.loop(0, PPASS)
    def _(p):
        plane = cid * PPASS + p
        valid = plane < NPL
        hidx = [lax.div(plane * PF4 + q * 16 + lanes, FXD)
                for q in range(PF4 // 16)]

        @pl.when(valid)
        def _():
            pltpu.sync_copy(zbuf, acc_sh.at[pl.ds(sid * ROWS, ROWS)])

        plsc.subcore_barrier()

        @pl.when(valid)
        def _():
            @pl.loop(0, NCH4, step=2)
            def _(j0):
                gh = [pltpu.async_copy(
                    h_hbm.at[plane].at[sidx.at[j0 + b]], hrow.at[b], gsem)
                    for b in range(2)]
                pltpu.sync_copy(
                    ex_hbm.at[pl.ds(sid * EW4 + j0 * CH, 2 * CH)], exbuf)
                for b in range(2):
                    gh[b].wait()
                sh = []
                for b in range(2):
                    @pl.loop(0, CH, step=4)
                    def _(i0, b=b):
                        for u in range(4):
                            exr = exbuf[b * CH + i0 + u]
                            for q in range(PF4 // 16):
                                sl = pl.ds(q * 16, 16)
                                wb = _bcast_lane(exr, hidx[q])
                                srow[b, i0 + u, sl] = (
                                    hrow[b, i0 + u, sl] * wb)

                    sh.append(pltpu.async_copy(
                        srow.at[b], acc_sh.at[didx.at[j0 + b]], gsem,
                        add=True))
                for b in range(2):
                    sh[b].wait()

                @pl.when(p == 0)
                def _():
                    for b in range(2):
                        pltpu.sync_copy(exbuf.at[pl.ds(b * CH, CH)],
                                        den_sh.at[didx.at[j0 + b]],
                                        add=True)

        plsc.subcore_barrier()

        @pl.when(valid)
        def _():
            pltpu.sync_copy(acc_sh.at[pl.ds(sid * ROWS, ROWS)],
                            agg_hbm.at[plane].at[pl.ds(sid * ROWS, ROWS)])

        @pl.when(jnp.logical_and(p == 0, cid == 0))
        def _():
            pltpu.sync_copy(den_sh.at[pl.ds(sid * ROWS, ROWS)],
                            den_hbm.at[pl.ds(sid * ROWS, ROWS)])

        plsc.subcore_barrier()


def _k5_body(agg_ref, den_ref, sel_ref, w3_ref, gb_ref, hgp_ref, dinv_ref):
    den = den_ref[...]
    rden = 1.0 / (den + 1e-16)
    deg = den[:, HEADS]
    dinv = jnp.where(deg > 0, lax.rsqrt(deg), 0.0)
    rdenx = jnp.dot(rden, sel_ref[...], preferred_element_type=jnp.float32)
    hg = jnp.zeros((RB, D_GC), jnp.float32)
    for p in range(NPL):
        sl = slice(p * PF4, (p + 1) * PF4)
        h1p = jax.nn.relu(agg_ref[p] * rdenx[:, sl] + gb_ref[...][:, sl])
        hg = hg + jnp.dot(h1p, w3_ref[p],
                          preferred_element_type=jnp.float32)
    hgp = hg * dinv[:, None]
    for q in range(NPL6):
        hgp_ref[q] = hgp[:, q * PW6:(q + 1) * PW6]
    dinv_ref[...] = jnp.broadcast_to(dinv[:, None], (RB, 16))


def _k5(agg3, den_t, sel, wg3, gatb_row):
    return pl.pallas_call(
        _k5_body,
        grid=(GRID,),
        in_specs=[
            pl.BlockSpec((NPL, RB, PF4), lambda i: (0, i, 0)),
            pl.BlockSpec((RB, 16), lambda i: (i, 0)),
            pl.BlockSpec((16, DGP), lambda i: (0, 0)),
            pl.BlockSpec((NPL, PF4, D_GC), lambda i: (0, 0, 0)),
            pl.BlockSpec((1, DGP), lambda i: (0, 0)),
        ],
        out_specs=[
            pl.BlockSpec((NPL6, RB, PW6), lambda i: (0, i, 0)),
            pl.BlockSpec((RB, 16), lambda i: (i, 0)),
        ],
        out_shape=[
            jax.ShapeDtypeStruct((NPL6, NP, PW6), jnp.float32),
            jax.ShapeDtypeStruct((NP, 16), jnp.float32),
        ],
    )(agg3, den_t, sel, wg3, gatb_row)


def _k6(*args):
    return pl.kernel(
        _k6_body,
        out_type=jax.ShapeDtypeStruct((NPL6, NP, PW6), jnp.float32),
        scratch_types=[
            pltpu.VMEM((NCH4, CH), jnp.int32),
            pltpu.VMEM((NCH4, CH), jnp.int32),
            pltpu.VMEM((2, CH, PW6), jnp.float32),
            pltpu.VMEM((ROWS, PW6), jnp.float32),
            pltpu.VMEM_SHARED((NP, PW6), jnp.float32),
            pltpu.SemaphoreType.DMA,
        ],
        **_sc_params(),
    )(*args)


def _k6_body(h_hbm, src_hbm, dst_hbm, agg_hbm, sidx, didx, hrow, zbuf,
        acc_sh, gsem):
    cid = lax.axis_index("c")
    sid = lax.axis_index("s")

    @pl.loop(0, ROWS)
    def _(r):
        for q in range(PW6 // 16):
            zbuf[r, pl.ds(q * 16, 16)] = jnp.zeros((16,), jnp.float32)

    pltpu.sync_copy(src_hbm.at[sid], sidx)
    pltpu.sync_copy(dst_hbm.at[sid], didx)

    @pl.loop(0, 7)
    def _(p):
        plane = cid * 7 + p
        valid = plane < NPL6

        @pl.when(valid)
        def _():
            pltpu.sync_copy(zbuf, acc_sh.at[pl.ds(sid * ROWS, ROWS)])

        plsc.subcore_barrier()

        @pl.when(valid)
        def _():
            @pl.loop(0, NCH4, step=2)
            def _(j0):
                gh = [pltpu.async_copy(
                    h_hbm.at[plane].at[sidx.at[j0 + b]], hrow.at[b], gsem)
                    for b in range(2)]
                for b in range(2):
                    gh[b].wait()
                sh = [pltpu.async_copy(
                    hrow.at[b], acc_sh.at[didx.at[j0 + b]], gsem, add=True)
                    for b in range(2)]
                for b in range(2):
                    sh[b].wait()

        plsc.subcore_barrier()

        @pl.when(valid)
        def _():
            pltpu.sync_copy(acc_sh.at[pl.ds(sid * ROWS, ROWS)],
                            agg_hbm.at[plane].at[pl.ds(sid * ROWS, ROWS)])

        plsc.subcore_barrier()


def _k6b_body(agg_ref, dinv_ref, gb_ref, h2_ref):
    dinv = dinv_ref[...][:, :1]
    cat = jnp.concatenate([agg_ref[q] for q in range(NPL6)], axis=1)
    h2_ref[...] = jax.nn.relu(cat * dinv + gb_ref[...])


def _k6b(agg2, dinv16, gcnb_row):
    return pl.pallas_call(
        _k6b_body,
        grid=(GRID,),
        in_specs=[
            pl.BlockSpec((NPL6, RB, PW6), lambda i: (0, i, 0)),
            pl.BlockSpec((RB, 16), lambda i: (i, 0)),
            pl.BlockSpec((1, D_GC), lambda i: (0, 0)),
        ],
        out_specs=pl.BlockSpec((RB, D_GC), lambda i: (i, 0)),
        out_shape=jax.ShapeDtypeStruct((NP, D_GC), jnp.float32),
    )(agg2, dinv16, gcnb_row)


def _k7(*args):
    return pl.kernel(
        _k7_body,
        out_type=(jax.ShapeDtypeStruct((B, D_GC), jnp.float32),
                  jax.ShapeDtypeStruct((B, D_GC), jnp.float32)),
        scratch_types=[
            pltpu.VMEM((264, 16), jnp.int32),
            pltpu.VMEM((16, D_GC), jnp.float32),
            pltpu.VMEM((D_GC,), jnp.float32),
            pltpu.VMEM((D_GC,), jnp.float32),
        ],
        **_sc_params(),
    )(*args)


def _k7_body(h2_hbm, offs_hbm, gmp_hbm, gsm_hbm, offs_v, hblk, mx, sm):
    cid = lax.axis_index("c")
    sid = lax.axis_index("s")
    wid = sid * 2 + cid
    pltpu.sync_copy(offs_hbm, offs_v)

    @pl.loop(0, GPW)
    def _(gl):
        g = wid * GPW + gl
        s = jnp.max(offs_v[g])
        e = jnp.max(offs_v[g + 1])
        a = s - lax.rem(s, 8)
        nb = (e - a + 15) // 16

        for q in range(D_GC // 16):
            sl = pl.ds(q * 16, 16)
            mx[sl] = jnp.full((16,), -jnp.inf, jnp.float32)
            sm[sl] = jnp.zeros((16,), jnp.float32)

        @pl.loop(0, nb)
        def _(bk):
            row0 = a + bk * 16
            pltpu.sync_copy(h2_hbm.at[pl.ds(row0, 16)], hblk)

            @pl.loop(0, 16)
            def _(i):
                r = row0 + i
                valid = jnp.logical_and(r >= s, r < e)

                @pl.when(valid)
                def _():
                    for q in range(D_GC // 16):
                        sl = pl.ds(q * 16, 16)
                        v = hblk[i, sl]
                        mx[sl] = jnp.maximum(mx[sl], v)
                        sm[sl] = sm[sl] + v

        pltpu.sync_copy(mx, gmp_hbm.at[g])
        pltpu.sync_copy(sm, gsm_hbm.at[g])


def _tower_body(gmp_ref, gsm_ref, cnt_ref, te_ref, fcg1_W, fcg1_b, fcg2_W,
                fcg2_b, plm_W, plm_b, bn_g, bn_b, fc1_W, fc1_b, fc2_W, fc2_b,
                out_W, out_b, o_ref):
    gmp = gmp_ref[...][:, :D_GAT]
    gsm = gsm_ref[...][:, :D_GAT]
    rcnt = 1.0 / jnp.maximum(cnt_ref[...], 1.0)
    g = jnp.concatenate([gmp, gsm * rcnt[:, None]], axis=1)
    gg = jax.nn.relu(
        jnp.dot(g, fcg1_W[...], preferred_element_type=jnp.float32)
        + fcg1_b[...])
    gg = (jnp.dot(gg, fcg2_W[...], preferred_element_type=jnp.float32)
          + fcg2_b[...])
    xt = (jnp.dot(te_ref[...], plm_W[...], preferred_element_type=jnp.float32)
          + plm_b[...])
    mean = jnp.mean(xt, axis=0)
    var = jnp.var(xt, axis=0)
    xt = (xt - mean) / jnp.sqrt(var + 1e-5) * bn_g[...] + bn_b[...]
    xt = jax.nn.relu(xt)
    xc = jnp.concatenate([gg, xt], axis=1)
    xc = jax.nn.relu(
        jnp.dot(xc, fc1_W[...], preferred_element_type=jnp.float32)
        + fc1_b[...])
    xc = jax.nn.relu(
        jnp.dot(xc, fc2_W[...], preferred_element_type=jnp.float32)
        + fc2_b[...])
    o_ref[...] = (jnp.dot(xc, out_W[...], preferred_element_type=jnp.float32)
                  + out_b[...])


def _tower(gmp, gsm, cnt, te, fcg1_W, fcg1_b, fcg2_W, fcg2_b, plm_W, plm_b,
           bn_g, bn_b, fc1_W, fc1_b, fc2_W, fc2_b, out_W, out_b):
    return pl.pallas_call(
        _tower_body,
        out_shape=jax.ShapeDtypeStruct((B, 1), jnp.float32),
    )(gmp, gsm, cnt, te, fcg1_W, fcg1_b, fcg2_W, fcg2_b, plm_W, plm_b,
      bn_g, bn_b, fc1_W, fc1_b, fc2_W, fc2_b, out_W, out_b)


def kernel(x, edge_index, batch, target_embedding, gat_W, gat_a_src,
           gat_a_dst, gat_b, gcn_W, gcn_b, fcg1_W, fcg1_b, fcg2_W, fcg2_b,
           plm_W, plm_b, bn_g, bn_b, fc1_W, fc1_b, fc2_W, fc2_b, out_W,
           out_b):
    loops = jnp.arange(N, dtype=edge_index.dtype)
    sink = jnp.full((EP - E - N,), N, jnp.int32)
    src = jnp.concatenate([edge_index[0], loops, sink])
    dst = jnp.concatenate([edge_index[1], loops, sink])
    src2 = src.reshape(NWORK, NCH, CH)
    dst2 = dst.reshape(NWORK, NCH, CH)
    src3 = src.reshape(16, NCH4, CH)
    dst3 = dst.reshape(16, NCH4, CH)

    xp = jnp.zeros((NP, FXD), jnp.float32).at[:N].set(x)
    w_pad = jnp.pad(gat_W, ((0, 0), (0, DGP - D_GAT)))
    w3 = w_pad.reshape(FXD, NPL, PF4).transpose(1, 0, 2)
    head_of = jnp.arange(DGP) // FXD
    sel = (head_of[None, :] == jnp.arange(16)[:, None]).astype(jnp.float32)
    sel = sel * (jnp.arange(DGP) < D_GAT)[None, :].astype(jnp.float32)
    as_flat = jnp.pad(gat_a_src.reshape(D_GAT), (0, DGP - D_GAT))
    ad_flat = jnp.pad(gat_a_dst.reshape(D_GAT), (0, DGP - D_GAT))
    asx = w_pad @ (as_flat[:, None] * sel.T)
    adx = w_pad @ (ad_flat[:, None] * sel.T)
    gatb_row = jnp.pad(gat_b, (0, DGP - D_GAT)).reshape(1, DGP)
    gcnb_row = jnp.pad(gcn_b, (0, D_GC - D_GAT)).reshape(1, D_GC)
    wg_pad = jnp.pad(gcn_W, ((0, DGP - D_GAT), (0, D_GC - D_GAT)))
    wg3 = wg_pad.reshape(NPL, PF4, D_GC)

    counts = jnp.sum(
        (batch[None, :] == jnp.arange(B, dtype=batch.dtype)[:, None])
        .astype(jnp.int32), axis=1)
    tri = (jnp.arange(B)[:, None] <= jnp.arange(B)[None, :]).astype(
        jnp.float32)
    csum = jnp.dot(counts.astype(jnp.float32), tri)
    offsets = jnp.concatenate(
        [jnp.zeros((1,), jnp.int32), csum.astype(jnp.int32)])
    offs_b = jnp.zeros((264, 16), jnp.int32).at[:B + 1].set(
        jnp.broadcast_to(offsets[:, None], (B + 1, 16)))
    cnt = (offsets[1:] - offsets[:-1]).astype(jnp.float32)

    h3, a_t, b_t = _k1(xp, w3, asx, adx)
    ex_all = _k2(src2, dst2, a_t, b_t)
    agg3, den_t = _k4(h3, src3, dst3, ex_all)
    hgp3, dinv16 = _k5(agg3, den_t, sel, wg3, gatb_row)
    agg2 = _k6(hgp3, src3, dst3)
    h2 = _k6b(agg2, dinv16, gcnb_row)
    gmp, gsm = _k7(h2, offs_b)
    return _tower(gmp, gsm, cnt, target_embedding, fcg1_W, fcg1_b, fcg2_W,
                  fcg2_b, plm_W, plm_b, bn_g, bn_b, fc1_W, fc1_b, fc2_W,
                  fc2_b, out_W, out_b)


# speedup vs baseline: 1.0011x; 1.0011x over previous
import functools

import jax
import jax.numpy as jnp
from jax import lax
from jax.experimental import pallas as pl
from jax.experimental.pallas import tpu as pltpu
from jax.experimental.pallas import tpu_sc as plsc

N = 10000
E = 160000
B = 256
FXD = 78
HEADS = 10
D_GAT = FXD * HEADS
EMBED = 320
OUT_DIM = 128

NP = 10112
EP = 172032
NWORK = 32
EW = EP // NWORK
CH = 128
NCH = EW // CH
ROWS = NP // 16
NCH4 = EP // 16 // CH
EW4 = EP // 16
HPASS = HEADS // 2
FP = 80
D_PAD = HEADS * FP
DGP = 800
PF4 = 32
NPL = 25
PPASS = 13
D_GC = 832
PW6 = 64
NPL6 = 13
RB = 632
GRID = NP // RB
GPW = B // NWORK


def _sc_mesh():
    return plsc.VectorSubcoreMesh(core_axis_name="c", subcore_axis_name="s")


def _sc_params():
    import dataclasses
    cp = pltpu.CompilerParams(use_tc_tiling_on_sc=False)
    if "needs_layout_passes" in pltpu.CompilerParams.__dataclass_fields__:
        cp = dataclasses.replace(cp, needs_layout_passes=False)
    return dict(mesh=_sc_mesh(), compiler_params=cp)


def _k1_body(x_ref, w3_ref, asx_ref, adx_ref, h_ref, a_ref, b_ref):
    x = x_ref[...]
    for p in range(NPL):
        h_ref[p] = jnp.dot(x, w3_ref[p], preferred_element_type=jnp.float32)
    a_ref[...] = jnp.dot(x, asx_ref[...], preferred_element_type=jnp.float32)
    b_ref[...] = jnp.dot(x, adx_ref[...], preferred_element_type=jnp.float32)


def _k1(xp, w3, asx, adx):
    return pl.pallas_call(
        _k1_body,
        grid=(GRID,),
        in_specs=[
            pl.BlockSpec((RB, FXD), lambda i: (i, 0)),
            pl.BlockSpec((NPL, FXD, PF4), lambda i: (0, 0, 0)),
            pl.BlockSpec((FXD, 16), lambda i: (0, 0)),
            pl.BlockSpec((FXD, 16), lambda i: (0, 0)),
        ],
        out_specs=[
            pl.BlockSpec((NPL, RB, PF4), lambda i: (0, i, 0)),
            pl.BlockSpec((RB, 16), lambda i: (i, 0)),
            pl.BlockSpec((RB, 16), lambda i: (i, 0)),
        ],
        out_shape=[
            jax.ShapeDtypeStruct((NPL, NP, PF4), jnp.float32),
            jax.ShapeDtypeStruct((NP, 16), jnp.float32),
            jax.ShapeDtypeStruct((NP, 16), jnp.float32),
        ],
    )(xp, w3, asx, adx)


def _k2(*args):
    return pl.kernel(
        _k2_body,
        out_type=jax.ShapeDtypeStruct((EP, 16), jnp.float32),
        scratch_types=[
            pltpu.VMEM((NCH, CH), jnp.int32),
            pltpu.VMEM((NCH, CH), jnp.int32),
            pltpu.VMEM((CH, 16), jnp.float32),
            pltpu.VMEM((CH, 16), jnp.float32),
            pltpu.VMEM((CH, 16), jnp.float32),
        ],
        **_sc_params(),
    )(*args)


def _k2_body(src_hbm, dst_hbm, a_hbm, b_hbm, ex_hbm,
        sidx, didx, arow, brow, exrow):
    cid = lax.axis_index("c")
    sid = lax.axis_index("s")
    wid = sid * 2 + cid

    pltpu.sync_copy(src_hbm.at[wid], sidx)
    pltpu.sync_copy(dst_hbm.at[wid], didx)
    lanes = lax.iota(jnp.int32, 16)

    @pl.loop(0, NCH)
    def _(j):
        pltpu.sync_copy(a_hbm.at[sidx.at[j]], arow)
        pltpu.sync_copy(b_hbm.at[didx.at[j]], brow)

        @pl.loop(0, CH)
        def _(i):
            v = arow[i] + brow[i]
            al = jnp.where(v >= 0.0, v, 0.2 * v)
            ex = jnp.exp(al)
            exrow[i] = jnp.where(
                lanes < HEADS, ex, jnp.where(lanes == HEADS, 1.0, 0.0))

        pltpu.sync_copy(exrow, ex_hbm.at[pl.ds(wid * EW + j * CH, CH)])


def _bcast_lane(v16, idx):
    return lax.gather(
        v16, idx[:, None],
        lax.GatherDimensionNumbers(offset_dims=(), collapsed_slice_dims=(0,),
                                   start_index_map=(0,)),
        (1,), mode=lax.GatherScatterMode.PROMISE_IN_BOUNDS)


def _k4(*args):
    return pl.kernel(
        _k4_body,
        out_type=(jax.ShapeDtypeStruct((NPL, NP, PF4), jnp.float32),
                  jax.ShapeDtypeStruct((NP, 16), jnp.float32)),
        scratch_types=[
            pltpu.VMEM((NCH4, CH), jnp.int32),
            pltpu.VMEM((NCH4, CH), jnp.int32),
            pltpu.VMEM((2, CH, PF4), jnp.float32),
            pltpu.VMEM((2, CH, PF4), jnp.float32),
            pltpu.VMEM((2 * CH, 16), jnp.float32),
            pltpu.VMEM((ROWS, PF4), jnp.float32),
            pltpu.VMEM((ROWS, 16), jnp.float32),
            pltpu.VMEM_SHARED((NP, PF4), jnp.float32),
            pltpu.VMEM_SHARED((NP, 16), jnp.float32),
            pltpu.SemaphoreType.DMA,
        ],
        **_sc_params(),
    )(*args)


def _k4_body(h_hbm, src_hbm, dst_hbm, ex_hbm, agg_hbm, den_hbm,
        sidx, didx, hrow, srow, exbuf, zbuf, zbuf16, acc_sh, den_sh, gsem):
    cid = lax.axis_index("c")
    sid = lax.axis_index("s")

    @pl.loop(0, ROWS)
    def _(r):
        for q in range(PF4 // 16):
            zbuf[r, pl.ds(q * 16, 16)] = jnp.zeros((16,), jnp.float32)
        zbuf16[r] = jnp.zeros((16,), jnp.float32)

    pltpu.sync_copy(src_hbm.at[sid], sidx)
    pltpu.sync_copy(dst_hbm.at[sid], didx)
    pltpu.sync_copy(zbuf16, den_sh.at[pl.ds(sid * ROWS, ROWS)])

    lanes = lax.iota(jnp.int32, 16)

    @pl.loop(0, PPASS)
    def _(p):
        plane = cid * PPASS + p
        valid = plane < NPL
        hidx = [lax.div(plane * PF4 + q * 16 + lanes, FXD)
                for q in range(PF4 // 16)]

        @pl.when(valid)
        def _():
            pltpu.sync_copy(zbuf, acc_sh.at[pl.ds(sid * ROWS, ROWS)])

        plsc.subcore_barrier()

        @pl.when(valid)
        def _():
            @pl.loop(0, NCH4, step=2)
            def _(j0):
                gh = [pltpu.async_copy(
                    h_hbm.at[plane].at[sidx.at[j0 + b]], hrow.at[b], gsem)
                    for b in range(2)]
                pltpu.sync_copy(
                    ex_hbm.at[pl.ds(sid * EW4 + j0 * CH, 2 * CH)], exbuf)
                for b in range(2):
                    gh[b].wait()
                sh = []
                for b in range(2):
                    @pl.loop(0, CH)
                    def _(i, b=b):
                        exr = exbuf[b * CH + i]
                        for q in range(PF4 // 16):
                            sl = pl.ds(q * 16, 16)
                            wb = _bcast_lane(exr, hidx[q])
                            srow[b, i, sl] = hrow[b, i, sl] * wb

                    sh.append(pltpu.async_copy(
                        srow.at[b], acc_sh.at[didx.at[j0 + b]], gsem,
                        add=True))
                for b in range(2):
                    sh[b].wait()

                @pl.when(p == 0)
                def _():
                    for b in range(2):
                        pltpu.sync_copy(exbuf.at[pl.ds(b * CH, CH)],
                                        den_sh.at[didx.at[j0 + b]],
                                        add=True)

        plsc.subcore_barrier()

        @pl.when(valid)
        def _():
            pltpu.sync_copy(acc_sh.at[pl.ds(sid * ROWS, ROWS)],
                            agg_hbm.at[plane].at[pl.ds(sid * ROWS, ROWS)])

        @pl.when(jnp.logical_and(p == 0, cid == 0))
        def _():
            pltpu.sync_copy(den_sh.at[pl.ds(sid * ROWS, ROWS)],
                            den_hbm.at[pl.ds(sid * ROWS, ROWS)])

        plsc.subcore_barrier()


def _k5_body(agg_ref, den_ref, sel_ref, w3_ref, gb_ref, hgp_ref, dinv_ref):
    den = den_ref[...]
    rden = 1.0 / (den + 1e-16)
    deg = den[:, HEADS]
    dinv = jnp.where(deg > 0, lax.rsqrt(deg), 0.0)
    rdenx = jnp.dot(rden, sel_ref[...], preferred_element_type=jnp.float32)
    hg = jnp.zeros((RB, D_GC), jnp.float32)
    for p in range(NPL):
        sl = slice(p * PF4, (p + 1) * PF4)
        h1p = jax.nn.relu(agg_ref[p] * rdenx[:, sl] + gb_ref[...][:, sl])
        hg = hg + jnp.dot(h1p, w3_ref[p],
                          preferred_element_type=jnp.float32)
    hgp = hg * dinv[:, None]
    for q in range(NPL6):
        hgp_ref[q] = hgp[:, q * PW6:(q + 1) * PW6]
    dinv_ref[...] = jnp.broadcast_to(dinv[:, None], (RB, 16))


def _k5(agg3, den_t, sel, wg3, gatb_row):
    return pl.pallas_call(
        _k5_body,
        grid=(GRID,),
        in_specs=[
            pl.BlockSpec((NPL, RB, PF4), lambda i: (0, i, 0)),
            pl.BlockSpec((RB, 16), lambda i: (i, 0)),
            pl.BlockSpec((16, DGP), lambda i: (0, 0)),
            pl.BlockSpec((NPL, PF4, D_GC), lambda i: (0, 0, 0)),
            pl.BlockSpec((1, DGP), lambda i: (0, 0)),
        ],
        out_specs=[
            pl.BlockSpec((NPL6, RB, PW6), lambda i: (0, i, 0)),
            pl.BlockSpec((RB, 16), lambda i: (i, 0)),
        ],
        out_shape=[
            jax.ShapeDtypeStruct((NPL6, NP, PW6), jnp.float32),
            jax.ShapeDtypeStruct((NP, 16), jnp.float32),
        ],
    )(agg3, den_t, sel, wg3, gatb_row)


def _k6(*args):
    return pl.kernel(
        _k6_body,
        out_type=jax.ShapeDtypeStruct((NPL6, NP, PW6), jnp.float32),
        scratch_types=[
            pltpu.VMEM((NCH4, CH), jnp.int32),
            pltpu.VMEM((NCH4, CH), jnp.int32),
            pltpu.VMEM((2, CH, PW6), jnp.float32),
            pltpu.VMEM((ROWS, PW6), jnp.float32),
            pltpu.VMEM_SHARED((NP, PW6), jnp.float32),
            pltpu.SemaphoreType.DMA,
        ],
        **_sc_params(),
    )(*args)


def _k6_body(h_hbm, src_hbm, dst_hbm, agg_hbm, sidx, didx, hrow, zbuf,
        acc_sh, gsem):
    cid = lax.axis_index("c")
    sid = lax.axis_index("s")

    @pl.loop(0, ROWS)
    def _(r):
        for q in range(PW6 // 16):
            zbuf[r, pl.ds(q * 16, 16)] = jnp.zeros((16,), jnp.float32)

    pltpu.sync_copy(src_hbm.at[sid], sidx)
    pltpu.sync_copy(dst_hbm.at[sid], didx)

    @pl.loop(0, 7)
    def _(p):
        plane = cid * 7 + p
        valid = plane < NPL6

        @pl.when(valid)
        def _():
            pltpu.sync_copy(zbuf, acc_sh.at[pl.ds(sid * ROWS, ROWS)])

        plsc.subcore_barrier()

        @pl.when(valid)
        def _():
            @pl.loop(0, NCH4, step=2)
            def _(j0):
                gh = [pltpu.async_copy(
                    h_hbm.at[plane].at[sidx.at[j0 + b]], hrow.at[b], gsem)
                    for b in range(2)]
                for b in range(2):
                    gh[b].wait()
                sh = [pltpu.async_copy(
                    hrow.at[b], acc_sh.at[didx.at[j0 + b]], gsem, add=True)
                    for b in range(2)]
                for b in range(2):
                    sh[b].wait()

        plsc.subcore_barrier()

        @pl.when(valid)
        def _():
            pltpu.sync_copy(acc_sh.at[pl.ds(sid * ROWS, ROWS)],
                            agg_hbm.at[plane].at[pl.ds(sid * ROWS, ROWS)])

        plsc.subcore_barrier()


def _k6b_body(agg_ref, dinv_ref, gb_ref, h2_ref):
    dinv = dinv_ref[...][:, :1]
    cat = jnp.concatenate([agg_ref[q] for q in range(NPL6)], axis=1)
    h2_ref[...] = jax.nn.relu(cat * dinv + gb_ref[...])


def _k6b(agg2, dinv16, gcnb_row):
    return pl.pallas_call(
        _k6b_body,
        grid=(GRID,),
        in_specs=[
            pl.BlockSpec((NPL6, RB, PW6), lambda i: (0, i, 0)),
            pl.BlockSpec((RB, 16), lambda i: (i, 0)),
            pl.BlockSpec((1, D_GC), lambda i: (0, 0)),
        ],
        out_specs=pl.BlockSpec((RB, D_GC), lambda i: (i, 0)),
        out_shape=jax.ShapeDtypeStruct((NP, D_GC), jnp.float32),
    )(agg2, dinv16, gcnb_row)


def _k7(*args):
    return pl.kernel(
        _k7_body,
        out_type=(jax.ShapeDtypeStruct((B, D_GC), jnp.float32),
                  jax.ShapeDtypeStruct((B, D_GC), jnp.float32)),
        scratch_types=[
            pltpu.VMEM((264, 16), jnp.int32),
            pltpu.VMEM((16, D_GC), jnp.float32),
            pltpu.VMEM((D_GC,), jnp.float32),
            pltpu.VMEM((D_GC,), jnp.float32),
        ],
        **_sc_params(),
    )(*args)


def _k7_body(h2_hbm, offs_hbm, gmp_hbm, gsm_hbm, offs_v, hblk, mx, sm):
    cid = lax.axis_index("c")
    sid = lax.axis_index("s")
    wid = sid * 2 + cid
    pltpu.sync_copy(offs_hbm, offs_v)

    @pl.loop(0, GPW)
    def _(gl):
        g = wid * GPW + gl
        s = jnp.max(offs_v[g])
        e = jnp.max(offs_v[g + 1])
        a = s - lax.rem(s, 8)
        nb = (e - a + 15) // 16

        for q in range(D_GC // 16):
            sl = pl.ds(q * 16, 16)
            mx[sl] = jnp.full((16,), -jnp.inf, jnp.float32)
            sm[sl] = jnp.zeros((16,), jnp.float32)

        @pl.loop(0, nb)
        def _(bk):
            row0 = a + bk * 16
            pltpu.sync_copy(h2_hbm.at[pl.ds(row0, 16)], hblk)

            @pl.loop(0, 16)
            def _(i):
                r = row0 + i
                valid = jnp.logical_and(r >= s, r < e)

                @pl.when(valid)
                def _():
                    for q in range(D_GC // 16):
                        sl = pl.ds(q * 16, 16)
                        v = hblk[i, sl]
                        mx[sl] = jnp.maximum(mx[sl], v)
                        sm[sl] = sm[sl] + v

        pltpu.sync_copy(mx, gmp_hbm.at[g])
        pltpu.sync_copy(sm, gsm_hbm.at[g])


def _tower_body(gmp_ref, gsm_ref, cnt_ref, te_ref, fcg1_W, fcg1_b, fcg2_W,
                fcg2_b, plm_W, plm_b, bn_g, bn_b, fc1_W, fc1_b, fc2_W, fc2_b,
                out_W, out_b, o_ref):
    gmp = gmp_ref[...][:, :D_GAT]
    gsm = gsm_ref[...][:, :D_GAT]
    rcnt = 1.0 / jnp.maximum(cnt_ref[...], 1.0)
    g = jnp.concatenate([gmp, gsm * rcnt[:, None]], axis=1)
    gg = jax.nn.relu(
        jnp.dot(g, fcg1_W[...], preferred_element_type=jnp.float32)
        + fcg1_b[...])
    gg = (jnp.dot(gg, fcg2_W[...], preferred_element_type=jnp.float32)
          + fcg2_b[...])
    xt = (jnp.dot(te_ref[...], plm_W[...], preferred_element_type=jnp.float32)
          + plm_b[...])
    mean = jnp.mean(xt, axis=0)
    var = jnp.var(xt, axis=0)
    xt = (xt - mean) / jnp.sqrt(var + 1e-5) * bn_g[...] + bn_b[...]
    xt = jax.nn.relu(xt)
    xc = jnp.concatenate([gg, xt], axis=1)
    xc = jax.nn.relu(
        jnp.dot(xc, fc1_W[...], preferred_element_type=jnp.float32)
        + fc1_b[...])
    xc = jax.nn.relu(
        jnp.dot(xc, fc2_W[...], preferred_element_type=jnp.float32)
        + fc2_b[...])
    o_ref[...] = (jnp.dot(xc, out_W[...], preferred_element_type=jnp.float32)
                  + out_b[...])


def _tower(gmp, gsm, cnt, te, fcg1_W, fcg1_b, fcg2_W, fcg2_b, plm_W, plm_b,
           bn_g, bn_b, fc1_W, fc1_b, fc2_W, fc2_b, out_W, out_b):
    return pl.pallas_call(
        _tower_body,
        out_shape=jax.ShapeDtypeStruct((B, 1), jnp.float32),
    )(gmp, gsm, cnt, te, fcg1_W, fcg1_b, fcg2_W, fcg2_b, plm_W, plm_b,
      bn_g, bn_b, fc1_W, fc1_b, fc2_W, fc2_b, out_W, out_b)


def kernel(x, edge_index, batch, target_embedding, gat_W, gat_a_src,
           gat_a_dst, gat_b, gcn_W, gcn_b, fcg1_W, fcg1_b, fcg2_W, fcg2_b,
           plm_W, plm_b, bn_g, bn_b, fc1_W, fc1_b, fc2_W, fc2_b, out_W,
           out_b):
    loops = jnp.arange(N, dtype=edge_index.dtype)
    sink = jnp.full((EP - E - N,), N, jnp.int32)
    src = jnp.concatenate([edge_index[0], loops, sink])
    dst = jnp.concatenate([edge_index[1], loops, sink])
    src2 = src.reshape(NWORK, NCH, CH)
    dst2 = dst.reshape(NWORK, NCH, CH)
    src3 = src.reshape(16, NCH4, CH)
    dst3 = dst.reshape(16, NCH4, CH)

    xp = jnp.zeros((NP, FXD), jnp.float32).at[:N].set(x)
    w_pad = jnp.pad(gat_W, ((0, 0), (0, DGP - D_GAT)))
    w3 = w_pad.reshape(FXD, NPL, PF4).transpose(1, 0, 2)
    head_of = jnp.arange(DGP) // FXD
    sel = (head_of[None, :] == jnp.arange(16)[:, None]).astype(jnp.float32)
    sel = sel * (jnp.arange(DGP) < D_GAT)[None, :].astype(jnp.float32)
    as_flat = jnp.pad(gat_a_src.reshape(D_GAT), (0, DGP - D_GAT))
    ad_flat = jnp.pad(gat_a_dst.reshape(D_GAT), (0, DGP - D_GAT))
    asx = w_pad @ (as_flat[:, None] * sel.T)
    adx = w_pad @ (ad_flat[:, None] * sel.T)
    gatb_row = jnp.pad(gat_b, (0, DGP - D_GAT)).reshape(1, DGP)
    gcnb_row = jnp.pad(gcn_b, (0, D_GC - D_GAT)).reshape(1, D_GC)
    wg_pad = jnp.pad(gcn_W, ((0, DGP - D_GAT), (0, D_GC - D_GAT)))
    wg3 = wg_pad.reshape(NPL, PF4, D_GC)

    counts = jnp.sum(
        (batch[None, :] == jnp.arange(B, dtype=batch.dtype)[:, None])
        .astype(jnp.int32), axis=1)
    tri = (jnp.arange(B)[:, None] <= jnp.arange(B)[None, :]).astype(
        jnp.float32)
    csum = jnp.dot(counts.astype(jnp.float32), tri)
    offsets = jnp.concatenate(
        [jnp.zeros((1,), jnp.int32), csum.astype(jnp.int32)])
    offs_b = jnp.zeros((264, 16), jnp.int32).at[:B + 1].set(
        jnp.broadcast_to(offsets[:, None], (B + 1, 16)))
    cnt = (offsets[1:] - offsets[:-1]).astype(jnp.float32)

    h3, a_t, b_t = _k1(xp, w3, asx, adx)
    ex_all = _k2(src2, dst2, a_t, b_t)
    agg3, den_t = _k4(h3, src3, dst3, ex_all)
    hgp3, dinv16 = _k5(agg3, den_t, sel, wg3, gatb_row)
    agg2 = _k6(hgp3, src3, dst3)
    h2 = _k6b(agg2, dinv16, gcnb_row)
    gmp, gsm = _k7(h2, offs_b)
    return _tower(gmp, gsm, cnt, target_embedding, fcg1_W, fcg1_b, fcg2_W,
                  fcg2_b, plm_W, plm_b, bn_g, bn_b, fc1_W, fc1_b, fc2_W,
                  fc2_b, out_W, out_b)
